# baseline (speedup 1.0000x reference)
import functools

import jax
import jax.numpy as jnp
from jax import lax
from jax.experimental import pallas as pl
from jax.experimental.pallas import tpu as pltpu
from jax.experimental.pallas import tpu_sc as plsc

N = 100000
K = 128
CH = 400
NCH = N // CH
NW = 32
GROUPS = CH // 16

_mesh = plsc.VectorSubcoreMesh(core_axis_name="c", subcore_axis_name="s")


@functools.partial(
    pl.kernel,
    mesh=_mesh,
    out_type=jax.ShapeDtypeStruct((N * K,), jnp.float32),
    scratch_types=[
        pltpu.VMEM((CH,), jnp.int32),
        pltpu.VMEM((CH * K,), jnp.float32),
    ],
    compiler_params=pltpu.CompilerParams(needs_layout_passes=False),
)
def _one_hot_sc(atom_hbm, out_hbm, idx_v, buf):
    info = plsc.get_sparse_core_info()
    wid = lax.axis_index("s") * info.num_cores + lax.axis_index("c")

    zeros = jnp.zeros((16,), jnp.float32)
    ones = jnp.ones((16,), jnp.float32)
    row_off = lax.iota(jnp.int32, 16) * K

    def zero_body(i, carry):
        buf[pl.ds(i * 16, 16)] = zeros
        return carry

    lax.fori_loop(0, CH * K // 16, zero_body, 0)

    n_mine = (NCH - wid + NW - 1) // NW

    def chunk_body(ci, carry):
        chunk = wid + ci * NW
        pltpu.sync_copy(atom_hbm.at[pl.ds(chunk * CH, CH)], idx_v)
        for g in range(GROUPS):
            cols = idx_v[pl.ds(g * 16, 16)]
            flat = row_off + cols + g * 16 * K
            plsc.store_scatter(buf, [flat], ones)
        pltpu.sync_copy(buf, out_hbm.at[pl.ds(chunk * CH * K, CH * K)])
        for g in range(GROUPS):
            cols = idx_v[pl.ds(g * 16, 16)]
            flat = row_off + cols + g * 16 * K
            plsc.store_scatter(buf, [flat], zeros)
        return carry

    lax.fori_loop(0, n_mine, chunk_body, 0)


def kernel(atom_type, pos):
    del pos
    out = _one_hot_sc(atom_type.astype(jnp.int32))
    return out.reshape(N, K)


# speedup vs baseline: 1.3696x; 1.3696x over previous
import functools

import jax
import jax.numpy as jnp
from jax import lax
from jax.experimental import pallas as pl
from jax.experimental.pallas import tpu as pltpu
from jax.experimental.pallas import tpu_sc as plsc

N = 100000
K = 128
CH = 400
CHK = CH * K
NCH = N // CH
NW = 32
GROUPS = CH // 16
MAXC = -(-NCH // NW)

_mesh = plsc.VectorSubcoreMesh(core_axis_name="c", subcore_axis_name="s")


def _scatter_groups(buf, idx_v, vals, row_off):
    for g in range(GROUPS):
        cols = idx_v[pl.ds(g * 16, 16)]
        flat = row_off + cols + g * 16 * K
        plsc.store_scatter(buf, [flat], vals)


@functools.partial(
    pl.kernel,
    mesh=_mesh,
    out_type=jax.ShapeDtypeStruct((N * K,), jnp.float32),
    scratch_types=[
        pltpu.VMEM((CHK,), jnp.float32),
        pltpu.VMEM((CHK,), jnp.float32),
        pltpu.VMEM((CH,), jnp.int32),
        pltpu.VMEM((CH,), jnp.int32),
        pltpu.VMEM((CH,), jnp.int32),
        pltpu.VMEM((CH,), jnp.int32),
        pltpu.SemaphoreType.DMA,
        pltpu.SemaphoreType.DMA,
        pltpu.SemaphoreType.DMA,
        pltpu.SemaphoreType.DMA,
        pltpu.SemaphoreType.DMA,
        pltpu.SemaphoreType.DMA,
    ],
    compiler_params=pltpu.CompilerParams(needs_layout_passes=False),
)
def _one_hot_sc(atom_hbm, out_hbm, buf0, buf1, i0, i1, i2, i3,
                os0, os1, is0, is1, is2, is3):
    info = plsc.get_sparse_core_info()
    wid = lax.axis_index("s") * info.num_cores + lax.axis_index("c")

    bufs = (buf0, buf1)
    idxs = (i0, i1, i2, i3)
    outsems = (os0, os1)
    idxsems = (is0, is1, is2, is3)

    zvec = jnp.zeros((16,), jnp.float32)
    ovec = jnp.ones((16,), jnp.float32)
    row_off = lax.iota(jnp.int32, 16) * K

    n_mine = (NCH - wid + NW - 1) // NW

    for ci in range(2):
        chunk = wid + ci * NW
        pltpu.async_copy(atom_hbm.at[pl.ds(chunk * CH, CH)],
                         idxs[ci], idxsems[ci])

    def _zero_body(i, carry):
        buf0[pl.ds(i * 16, 16)] = zvec
        buf1[pl.ds(i * 16, 16)] = zvec
        return carry

    lax.fori_loop(0, CHK // 16, _zero_body, 0, unroll=16)

    def _chunk_body(ci):
        b = ci % 2
        s = ci % 4
        buf = bufs[b]
        chunk = wid + ci * NW
        if ci >= 2:
            pltpu.make_async_copy(buf, out_hbm.at[pl.ds(0, CHK)],
                                  outsems[b]).wait()
            _scatter_groups(buf, idxs[(ci - 2) % 4], zvec, row_off)
        pltpu.make_async_copy(atom_hbm.at[pl.ds(0, CH)], idxs[s],
                              idxsems[s]).wait()
        _scatter_groups(buf, idxs[s], ovec, row_off)
        pltpu.async_copy(buf, out_hbm.at[pl.ds(chunk * CHK, CHK)], outsems[b])
        cj = ci + 2
        if cj < MAXC:
            @pl.when(cj < n_mine)
            def _():
                chunk2 = wid + cj * NW
                pltpu.async_copy(atom_hbm.at[pl.ds(chunk2 * CH, CH)],
                                 idxs[cj % 4], idxsems[cj % 4])

    for ci in range(MAXC):
        if ci < 2:
            _chunk_body(ci)
        else:
            pl.when(ci < n_mine)(lambda ci=ci: _chunk_body(ci))

    pltpu.make_async_copy(buf0, out_hbm.at[pl.ds(0, CHK)], os0).wait()
    pltpu.make_async_copy(buf1, out_hbm.at[pl.ds(0, CHK)], os1).wait()


def kernel(atom_type, pos):
    del pos
    out = _one_hot_sc(atom_type.astype(jnp.int32))
    return out.reshape(N, K)


# speedup vs baseline: 1.4329x; 1.0462x over previous
import functools

import jax
import jax.numpy as jnp
from jax import lax
from jax.experimental import pallas as pl
from jax.experimental.pallas import tpu as pltpu
from jax.experimental.pallas import tpu_sc as plsc

N = 100000
K = 128
CH = 400
CHK = CH * K
NCH = N // CH
NW = 32
GROUPS = CH // 16
MAXC = -(-NCH // NW)

_mesh = plsc.VectorSubcoreMesh(core_axis_name="c", subcore_axis_name="s")


@functools.partial(
    pl.kernel,
    mesh=_mesh,
    out_type=jax.ShapeDtypeStruct((N * K,), jnp.float32),
    scratch_types=[
        pltpu.VMEM((CHK,), jnp.float32),
        pltpu.VMEM((CHK,), jnp.float32),
        pltpu.VMEM((CH,), jnp.int32),
        pltpu.VMEM((CH,), jnp.int32),
        pltpu.VMEM((CH,), jnp.int32),
        pltpu.VMEM((CH,), jnp.int32),
        pltpu.SemaphoreType.DMA,
        pltpu.SemaphoreType.DMA,
        pltpu.SemaphoreType.DMA,
        pltpu.SemaphoreType.DMA,
    ],
    compiler_params=pltpu.CompilerParams(needs_layout_passes=False),
)
def _one_hot_sc(atom_hbm, out_hbm, buf0, buf1, idx0, idx1, fi0, fi1,
                os0, os1, is0, is1):
    info = plsc.get_sparse_core_info()
    wid = lax.axis_index("s") * info.num_cores + lax.axis_index("c")

    bufs = (buf0, buf1)
    idxs = (idx0, idx1)
    fis = (fi0, fi1)
    outsems = (os0, os1)
    idxsems = (is0, is1)

    zvec = jnp.zeros((16,), jnp.float32)
    ovec = jnp.ones((16,), jnp.float32)
    row_off = lax.iota(jnp.int32, 16) * K

    n_mine = (NCH - wid + NW - 1) // NW

    def _prefetch_idx(ci, b):
        chunk = wid + ci * NW
        pltpu.async_copy(atom_hbm.at[pl.ds(chunk * CH, CH)],
                         idxs[b], idxsems[b])

    def _fill(b):
        buf, idx_v, fi = bufs[b], idxs[b], fis[b]

        def body(g, carry):
            base = row_off + g * (16 * K)
            flat = base + idx_v[pl.ds(g * 16, 16)]
            fi[pl.ds(g * 16, 16)] = flat
            plsc.store_scatter(buf, [flat], ovec)
            return carry

        lax.fori_loop(0, GROUPS, body, 0, unroll=5)

    def _clear(b):
        buf, fi = bufs[b], fis[b]

        def body(g, carry):
            plsc.store_scatter(buf, [fi[pl.ds(g * 16, 16)]], zvec)
            return carry

        lax.fori_loop(0, GROUPS, body, 0, unroll=5)

    _prefetch_idx(0, 0)
    _prefetch_idx(1, 1)

    def _zero_body(i, carry):
        buf0[pl.ds(i * 16, 16)] = zvec
        buf1[pl.ds(i * 16, 16)] = zvec
        return carry

    lax.fori_loop(0, CHK // 16, _zero_body, 0, unroll=8)

    for b in range(2):
        pltpu.make_async_copy(atom_hbm.at[pl.ds(0, CH)], idxs[b],
                              idxsems[b]).wait()
        _fill(b)
        chunk = wid + b * NW
        pltpu.async_copy(bufs[b], out_hbm.at[pl.ds(chunk * CHK, CHK)],
                         outsems[b])
        _prefetch_idx(b + 2, b)

    def _pair_body(i2, carry):
        for b in range(2):
            ci = 2 * i2 + b

            @pl.when(ci < n_mine)
            def _():
                pltpu.make_async_copy(bufs[b], out_hbm.at[pl.ds(0, CHK)],
                                      outsems[b]).wait()
                _clear(b)
                pltpu.make_async_copy(atom_hbm.at[pl.ds(0, CH)], idxs[b],
                                      idxsems[b]).wait()
                _fill(b)
                chunk = wid + ci * NW
                pltpu.async_copy(bufs[b],
                                 out_hbm.at[pl.ds(chunk * CHK, CHK)],
                                 outsems[b])

                @pl.when(ci + 2 < n_mine)
                def _():
                    _prefetch_idx(ci + 2, b)
        return carry

    lax.fori_loop(1, (MAXC + 1) // 2, _pair_body, 0)

    pltpu.make_async_copy(buf0, out_hbm.at[pl.ds(0, CHK)], os0).wait()
    pltpu.make_async_copy(buf1, out_hbm.at[pl.ds(0, CHK)], os1).wait()


def kernel(atom_type, pos):
    del pos
    out = _one_hot_sc(atom_type.astype(jnp.int32))
    return out.reshape(N, K)


# speedup vs baseline: 1.5384x; 1.0736x over previous
import functools

import jax
import jax.numpy as jnp
from jax import lax
from jax.experimental import pallas as pl
from jax.experimental.pallas import tpu as pltpu
from jax.experimental.pallas import tpu_sc as plsc

N = 100000
K = 128
CH = 160
CHK = CH * K
NCH = N // CH
NW = 32
GROUPS = CH // 16
MAXC = -(-NCH // NW)

_mesh = plsc.VectorSubcoreMesh(core_axis_name="c", subcore_axis_name="s")


@functools.partial(
    pl.kernel,
    mesh=_mesh,
    out_type=jax.ShapeDtypeStruct((N * K,), jnp.float32),
    scratch_types=[
        pltpu.VMEM((CHK,), jnp.float32),
        pltpu.VMEM((CHK,), jnp.float32),
        pltpu.VMEM((CH,), jnp.int32),
        pltpu.VMEM((CH,), jnp.int32),
        pltpu.VMEM((CH,), jnp.int32),
        pltpu.VMEM((CH,), jnp.int32),
        pltpu.SemaphoreType.DMA,
        pltpu.SemaphoreType.DMA,
        pltpu.SemaphoreType.DMA,
        pltpu.SemaphoreType.DMA,
    ],
    compiler_params=pltpu.CompilerParams(needs_layout_passes=False),
)
def _one_hot_sc(atom_hbm, out_hbm, buf0, buf1, idx0, idx1, fi0, fi1,
                os0, os1, is0, is1):
    info = plsc.get_sparse_core_info()
    wid = lax.axis_index("s") * info.num_cores + lax.axis_index("c")

    bufs = (buf0, buf1)
    idxs = (idx0, idx1)
    fis = (fi0, fi1)
    outsems = (os0, os1)
    idxsems = (is0, is1)

    zvec = jnp.zeros((16,), jnp.float32)
    ovec = jnp.ones((16,), jnp.float32)
    row_off = lax.iota(jnp.int32, 16) * K

    n_mine = (NCH - wid + NW - 1) // NW

    def _prefetch_idx(ci, b):
        chunk = wid + ci * NW
        pltpu.async_copy(atom_hbm.at[pl.ds(chunk * CH, CH)],
                         idxs[b], idxsems[b])

    def _fill(b):
        buf, idx_v, fi = bufs[b], idxs[b], fis[b]

        def body(g, carry):
            base = row_off + g * (16 * K)
            flat = base + idx_v[pl.ds(g * 16, 16)]
            fi[pl.ds(g * 16, 16)] = flat
            plsc.store_scatter(buf, [flat], ovec)
            return carry

        lax.fori_loop(0, GROUPS, body, 0, unroll=5)

    def _clear(b):
        buf, fi = bufs[b], fis[b]

        def body(g, carry):
            plsc.store_scatter(buf, [fi[pl.ds(g * 16, 16)]], zvec)
            return carry

        lax.fori_loop(0, GROUPS, body, 0, unroll=5)

    _prefetch_idx(0, 0)
    _prefetch_idx(1, 1)

    for b in range(2):
        buf = bufs[b]

        def _zero_body(i, carry):
            buf[pl.ds(i * 16, 16)] = zvec
            return carry

        lax.fori_loop(0, CHK // 16, _zero_body, 0, unroll=8)
        pltpu.make_async_copy(atom_hbm.at[pl.ds(0, CH)], idxs[b],
                              idxsems[b]).wait()
        _fill(b)
        chunk = wid + b * NW
        pltpu.async_copy(bufs[b], out_hbm.at[pl.ds(chunk * CHK, CHK)],
                         outsems[b])
        _prefetch_idx(b + 2, b)

    def _pair_body(i2, carry):
        for b in range(2):
            ci = 2 * i2 + b

            @pl.when(ci < n_mine)
            def _():
                pltpu.make_async_copy(bufs[b], out_hbm.at[pl.ds(0, CHK)],
                                      outsems[b]).wait()
                _clear(b)
                pltpu.make_async_copy(atom_hbm.at[pl.ds(0, CH)], idxs[b],
                                      idxsems[b]).wait()
                _fill(b)
                chunk = wid + ci * NW
                pltpu.async_copy(bufs[b],
                                 out_hbm.at[pl.ds(chunk * CHK, CHK)],
                                 outsems[b])

                @pl.when(ci + 2 < n_mine)
                def _():
                    _prefetch_idx(ci + 2, b)
        return carry

    lax.fori_loop(1, (MAXC + 1) // 2, _pair_body, 0)

    pltpu.make_async_copy(buf0, out_hbm.at[pl.ds(0, CHK)], os0).wait()
    pltpu.make_async_copy(buf1, out_hbm.at[pl.ds(0, CHK)], os1).wait()


def kernel(atom_type, pos):
    del pos
    out = _one_hot_sc(atom_type.astype(jnp.int32))
    return out.reshape(N, K)
